# Initial kernel scaffold; baseline (speedup 1.0000x reference)
import jax
import jax.numpy as jnp
from jax.experimental import pallas as pl


def kernel(users, pos_items, neg_items, edge_index, edge_weight, user_gender, user_age_bucket, item_cat, user_emb, item_emb, gender_emb, age_emb, cat_emb):
    raise NotImplementedError("write your pallas kernel here")


# speedup vs baseline: 1.4673x; 1.4673x over previous
import functools

import jax
import jax.numpy as jnp
from jax import lax
from jax.experimental import pallas as pl
from jax.experimental.pallas import tpu as pltpu
from jax.experimental.pallas import tpu_sc as plsc

NU = 25000
NI = 25000
NN = NU + NI
NNP = 51200
NE = 800000
NEP = 819200
D = 64
H = 32
B = 4096
NL = 3
DECAY = 1e-4

NBLK = 128
NUB = 196
EPT = NEP // 16
SB = 256
NSB = EPT // SB
RPT = NNP // 16
SPT = B // 16

_f32 = jnp.float32
_i32 = jnp.int32


def _iota16():
  return lax.iota(_i32, 16)


def _sc_body(users_h, pos_h, neg_h, src_h, dst2_h, w_h,
             ug_h, ua_h, ic_h, ue_h, ie_h, ge_h, ae_h, ce_h,
             xs_h, mean_h, reg_h,
             acc, gtab, atab, ctab, ublock, outblock, rows,
             srcb, dstb, wb, gidx, aidx, sem):
  cid = lax.axis_index("c")
  sid = lax.axis_index("s")
  col_base = cid * H

  pltpu.sync_copy(ge_h, gtab)
  pltpu.sync_copy(ae_h, atab)
  pltpu.sync_copy(ce_h, ctab)

  def _build_user(b):
    n0 = jnp.minimum(NBLK * b, NU - NBLK)
    pltpu.sync_copy(ue_h.at[pl.ds(n0, NBLK)], ublock)
    pltpu.sync_copy(ug_h.at[pl.ds(n0, NBLK)], gidx)
    pltpu.sync_copy(ua_h.at[pl.ds(n0, NBLK)], aidx)

    def _grp(g, _):
      rowv = _iota16() + 16 * g
      gv = gidx[pl.ds(16 * g, 16)]
      av = aidx[pl.ds(16 * g, 16)]

      def _col(d, __):
        colv = jnp.full((16,), col_base + d, _i32)
        uv = plsc.load_gather(ublock, [rowv, colv])
        gvv = plsc.load_gather(gtab, [gv, colv])
        avv = plsc.load_gather(atab, [av, colv])
        plsc.store_scatter(outblock, [rowv, jnp.full((16,), d, _i32)],
                           uv + gvv + avv)
        return __
      lax.fori_loop(0, H, _col, None)
      return _
    lax.fori_loop(0, NBLK // 16, _grp, None)
    pltpu.sync_copy(outblock, xs_h.at[0, cid, pl.ds(n0, NBLK)])

  def _build_item(b):
    n0 = jnp.minimum(NBLK * b, NI - NBLK)
    pltpu.sync_copy(ie_h.at[pl.ds(n0, NBLK)], ublock)
    pltpu.sync_copy(ic_h.at[pl.ds(n0, NBLK)], gidx)

    def _grp(g, _):
      rowv = _iota16() + 16 * g
      cv = gidx[pl.ds(16 * g, 16)]

      def _col(d, __):
        colv = jnp.full((16,), col_base + d, _i32)
        iv = plsc.load_gather(ublock, [rowv, colv])
        cvv = plsc.load_gather(ctab, [cv, colv])
        plsc.store_scatter(outblock, [rowv, jnp.full((16,), d, _i32)],
                           iv + cvv)
        return __
      lax.fori_loop(0, H, _col, None)
      return _
    lax.fori_loop(0, NBLK // 16, _grp, None)
    pltpu.sync_copy(outblock, xs_h.at[0, cid, pl.ds(NU + n0, NBLK)])

  for k in range(13):
    b = sid + 16 * k

    @pl.when(b < NUB)
    def _():
      _build_user(b)

    @pl.when(b < NUB)
    def _():
      _build_item(b)

  plsc.subcore_barrier()

  def _zb(r, _):
    outblock[r, pl.ds(0, 16)] = jnp.zeros((16,), _f32)
    outblock[r, pl.ds(16, 16)] = jnp.zeros((16,), _f32)
    return _
  lax.fori_loop(0, NBLK, _zb, None)

  for l in range(NL):
    def _zero(z, _):
      pltpu.sync_copy(outblock, acc.at[pl.ds(sid * RPT + NBLK * z, NBLK)])
      return _
    lax.fori_loop(0, RPT // NBLK, _zero, None)
    plsc.subcore_barrier()

    def _superblock(s, _):
      e0 = sid * EPT + SB * s
      r0 = sid * (EPT // 128) + 2 * s
      pltpu.sync_copy(src_h.at[pl.ds(e0, SB)], srcb)
      pltpu.sync_copy(dst2_h.at[pl.ds(r0, 2)], dstb)
      pltpu.sync_copy(w_h.at[pl.ds(e0, SB)], wb)
      descs = []
      for j in range(2):
        descs.append(pltpu.async_copy(
            xs_h.at[l, cid].at[srcb.at[pl.ds(128 * j, 128)]],
            rows.at[pl.ds(128 * j, 128)], sem))
      for dsc in descs:
        dsc.wait()

      def _grp(eg, __):
        w16 = wb[pl.ds(16 * eg, 16)]
        rowv = _iota16() + 16 * eg

        def _col(dc, ___):
          colv = jnp.full((16,), dc, _i32)
          v = plsc.load_gather(rows, [rowv, colv])
          plsc.store_scatter(rows, [rowv, colv], v * w16)
          return ___
        lax.fori_loop(0, H, _col, None)
        return __
      lax.fori_loop(0, SB // 16, _grp, None)

      for j in range(2):
        pltpu.sync_copy(rows.at[pl.ds(128 * j, 128)],
                        acc.at[dstb.at[j]], add=True)
      return _
    lax.fori_loop(0, NSB, _superblock, None)
    plsc.subcore_barrier()

    def _cpout(z, _):
      rr = sid * RPT + NBLK * z
      pltpu.sync_copy(acc.at[pl.ds(rr, NBLK)],
                      xs_h.at[l + 1, cid, pl.ds(rr, NBLK)])
      return _
    lax.fori_loop(0, RPT // NBLK, _cpout, None)
    plsc.subcore_barrier()

  for ridx, idx_h in enumerate((users_h, pos_h, neg_h)):
    for t in range(SPT // 128):
      s0 = sid * SPT + 128 * t
      pltpu.sync_copy(idx_h.at[pl.ds(s0, 128)], gidx)
      if ridx > 0:
        def _off(i, _):
          v = gidx[pl.ds(16 * i, 16)]
          gidx[pl.ds(16 * i, 16)] = v + NU
          return _
        lax.fori_loop(0, 8, _off, None)
      for pair in range(2):
        descs = []
        for j in range(2):
          descs.append(pltpu.async_copy(
              xs_h.at[2 * pair + j, cid].at[gidx],
              rows.at[pl.ds(128 * j, 128)], sem))
        for dsc in descs:
          dsc.wait()

        def _mrow(r, _):
          for h2 in range(2):
            sl = pl.ds(16 * h2, 16)
            v = rows[r, sl] + rows[128 + r, sl]
            if pair == 0:
              outblock[r, sl] = v
            else:
              outblock[r, sl] = (outblock[r, sl] + v) * 0.25
          return _
        lax.fori_loop(0, 128, _mrow, None)
      pltpu.sync_copy(outblock, mean_h.at[ridx, cid, pl.ds(s0, 128)])

  for ridx, (idx_h, tbl_h) in enumerate(((users_h, ue_h),
                                         (pos_h, ie_h),
                                         (neg_h, ie_h))):
    s0 = cid * (B // 2) + sid * 128
    pltpu.sync_copy(idx_h.at[pl.ds(s0, 128)], gidx)
    pltpu.async_copy(tbl_h.at[gidx], ublock, sem).wait()
    pltpu.sync_copy(ublock, reg_h.at[ridx, pl.ds(s0, 128)])


_sc_forward = pl.kernel(
    _sc_body,
    out_type=(
        jax.ShapeDtypeStruct((NL + 1, 2, NNP, H), _f32),
        jax.ShapeDtypeStruct((3, 2, B, H), _f32),
        jax.ShapeDtypeStruct((3, B, D), _f32),
    ),
    mesh=plsc.VectorSubcoreMesh(core_axis_name="c", subcore_axis_name="s",
                                num_cores=2, num_subcores=16),
    compiler_params=pltpu.CompilerParams(needs_layout_passes=False,
                                         use_tc_tiling_on_sc=False),
    scratch_types=[
        pltpu.VMEM_SHARED((NNP, H), _f32),
        pltpu.VMEM((3, D), _f32),
        pltpu.VMEM((8, D), _f32),
        pltpu.VMEM((11, D), _f32),
        pltpu.VMEM((NBLK, D), _f32),
        pltpu.VMEM((NBLK, H), _f32),
        pltpu.VMEM((SB, H), _f32),
        pltpu.VMEM((SB,), _i32),
        pltpu.VMEM((2, 128), _i32),
        pltpu.VMEM((SB,), _f32),
        pltpu.VMEM((NBLK,), _i32),
        pltpu.VMEM((NBLK,), _i32),
        pltpu.SemaphoreType.DMA,
    ],
)


def _loss_body(mean_ref, reg_ref, out_ref):
  u = mean_ref[0]
  pi = mean_ref[1]
  ni = mean_ref[2]
  ps = jnp.sum(u * pi, axis=(0, 2))
  ns = jnp.sum(u * ni, axis=(0, 2))
  x = ps - ns
  bpr = -jnp.mean(jnp.minimum(x, 0.0) - jnp.log1p(jnp.exp(-jnp.abs(x))))
  r = reg_ref[...]
  reg = jnp.sum(r * r) / B
  out_ref[...] = jnp.reshape(bpr + DECAY * reg, (1, 1))


_tc_loss = pl.pallas_call(
    _loss_body,
    out_shape=jax.ShapeDtypeStruct((1, 1), _f32),
)


@jax.jit
def kernel(users, pos_items, neg_items, edge_index, edge_weight,
           user_gender, user_age_bucket, item_cat,
           user_emb, item_emb, gender_emb, age_emb, cat_emb):
  dst = edge_index[0].astype(_i32)
  src = edge_index[1].astype(_i32)
  pad = NEP - NE
  pad_idx = jnp.arange(pad, dtype=_i32)
  src1 = jnp.concatenate([src, pad_idx % NN])
  dst1 = jnp.concatenate([dst, NN + pad_idx % (NNP - NN)])
  w1 = jnp.concatenate([edge_weight.astype(_f32), jnp.zeros((pad,), _f32)])
  dst2 = dst1.reshape(NEP // 128, 128)

  _, mean_rows, reg_rows = _sc_forward(
      users.astype(_i32), pos_items.astype(_i32), neg_items.astype(_i32),
      src1, dst2, w1,
      user_gender.astype(_i32), user_age_bucket.astype(_i32),
      item_cat.astype(_i32),
      user_emb.astype(_f32), item_emb.astype(_f32),
      gender_emb.astype(_f32), age_emb.astype(_f32), cat_emb.astype(_f32))

  loss = _tc_loss(mean_rows, reg_rows)
  return jnp.reshape(loss, ())


# speedup vs baseline: 1.7906x; 1.2204x over previous
import functools

import jax
import jax.numpy as jnp
from jax import lax
from jax.experimental import pallas as pl
from jax.experimental.pallas import tpu as pltpu
from jax.experimental.pallas import tpu_sc as plsc

NU = 25000
NI = 25000
NN = NU + NI
NNP = 51200
NE = 800000
NEP = 819200
D = 64
H = 32
B = 4096
NL = 3
DECAY = 1e-4

NBLK = 64
NUB = 391
EPT = NEP // 16
SB = 256
NSB = EPT // SB
RPT = NNP // 16
SPT = B // 16

_f32 = jnp.float32
_i32 = jnp.int32


def _iota16():
  return lax.iota(_i32, 16)


def _sc_body(users_h, pos_h, neg_h, src_h, dst2_h, w_h,
             ug_h, ua_h, ic_h, ue_h, ie_h, ge_h, ae_h, ce_h,
             xs_h, mean_h, reg_h,
             acc, gtab, atab, ctab, ublock, outblock, rows0, rows1,
             srcb0, srcb1, srcb2, srcb3, dstb0, dstb1, dstb2, dstb3,
             wb0, wb1, wb2, wb3,
             esem0, esem1, esem2, esem3, gsem0, gsem1, usem):
  cid = lax.axis_index("c")
  sid = lax.axis_index("s")
  col_base = cid * H
  rows_ = (rows0, rows1)
  srcb_ = (srcb0, srcb1, srcb2, srcb3)
  dstb_ = (dstb0, dstb1, dstb2, dstb3)
  wb_ = (wb0, wb1, wb2, wb3)
  esem_ = (esem0, esem1, esem2, esem3)
  gsem_ = (gsem0, gsem1)

  pltpu.sync_copy(ge_h, gtab)
  pltpu.sync_copy(ae_h, atab)
  pltpu.sync_copy(ce_h, ctab)

  def _build_user(b):
    n0 = jnp.minimum(NBLK * b, NU - NBLK)
    d1 = pltpu.async_copy(ue_h.at[pl.ds(n0, NBLK)], ublock, usem)
    d2 = pltpu.async_copy(ug_h.at[pl.ds(n0, NBLK)], gidx_b, usem)
    d3 = pltpu.async_copy(ua_h.at[pl.ds(n0, NBLK)], aidx_b, usem)
    d1.wait(); d2.wait(); d3.wait()

    def _grp(g, _):
      rowv = _iota16() + 16 * g
      gv = gidx_b[pl.ds(16 * g, 16)]
      av = aidx_b[pl.ds(16 * g, 16)]
      for d in range(H):
        colv = jnp.full((16,), col_base + d, _i32)
        uv = plsc.load_gather(ublock, [rowv, colv])
        gvv = plsc.load_gather(gtab, [gv, colv])
        avv = plsc.load_gather(atab, [av, colv])
        plsc.store_scatter(outblock, [rowv, jnp.full((16,), d, _i32)],
                           uv + gvv + avv)
      return _
    lax.fori_loop(0, NBLK // 16, _grp, None)
    pltpu.sync_copy(outblock, xs_h.at[0, cid, pl.ds(n0, NBLK)])

  def _build_item(b):
    n0 = jnp.minimum(NBLK * b, NI - NBLK)
    d1 = pltpu.async_copy(ie_h.at[pl.ds(n0, NBLK)], ublock, usem)
    d2 = pltpu.async_copy(ic_h.at[pl.ds(n0, NBLK)], gidx_b, usem)
    d1.wait(); d2.wait()

    def _grp(g, _):
      rowv = _iota16() + 16 * g
      cv = gidx_b[pl.ds(16 * g, 16)]
      for d in range(H):
        colv = jnp.full((16,), col_base + d, _i32)
        iv = plsc.load_gather(ublock, [rowv, colv])
        cvv = plsc.load_gather(ctab, [cv, colv])
        plsc.store_scatter(outblock, [rowv, jnp.full((16,), d, _i32)],
                           iv + cvv)
      return _
    lax.fori_loop(0, NBLK // 16, _grp, None)
    pltpu.sync_copy(outblock, xs_h.at[0, cid, pl.ds(NU + n0, NBLK)])

  gidx_b = srcb0.at[pl.ds(0, NBLK)]
  aidx_b = srcb0.at[pl.ds(NBLK, NBLK)]

  def _build_k(k, _):
    b = sid + 16 * k

    @pl.when(b < NUB)
    def _():
      _build_user(b)
      _build_item(b)
    return _
  lax.fori_loop(0, 25, _build_k, None)

  plsc.subcore_barrier()

  def _zb(r, _):
    outblock[r, pl.ds(0, 16)] = jnp.zeros((16,), _f32)
    outblock[r, pl.ds(16, 16)] = jnp.zeros((16,), _f32)
    return _
  lax.fori_loop(0, NBLK, _zb, None)

  def _fire_edges(s, e):
    e0 = sid * EPT + SB * s
    r0 = sid * (EPT // 128) + 2 * s
    da = pltpu.async_copy(src_h.at[pl.ds(e0, SB)], srcb_[e], esem_[e])
    db = pltpu.async_copy(dst2_h.at[pl.ds(r0, 2)], dstb_[e], esem_[e])
    dc = pltpu.async_copy(w_h.at[pl.ds(e0, SB)], wb_[e], esem_[e])
    return da, db, dc

  def _wait_edges(s, e):
    e0 = sid * EPT + SB * s
    r0 = sid * (EPT // 128) + 2 * s
    pltpu.make_async_copy(src_h.at[pl.ds(e0, SB)], srcb_[e], esem_[e]).wait()
    pltpu.make_async_copy(dst2_h.at[pl.ds(r0, 2)], dstb_[e], esem_[e]).wait()
    pltpu.make_async_copy(w_h.at[pl.ds(e0, SB)], wb_[e], esem_[e]).wait()

  def _fire_gathers(l, e, p):
    for j in range(2):
      pltpu.async_copy(
          xs_h.at[l, cid].at[srcb_[e].at[pl.ds(128 * j, 128)]],
          rows_[p].at[pl.ds(128 * j, 128)], gsem_[p])

  def _wait_gathers(l, e, p):
    for j in range(2):
      pltpu.make_async_copy(
          xs_h.at[l, cid].at[srcb_[e].at[pl.ds(128 * j, 128)]],
          rows_[p].at[pl.ds(128 * j, 128)], gsem_[p]).wait()

  for l in range(NL):
    zdescs = []
    for z in range(RPT // NBLK):
      zdescs.append(pltpu.async_copy(
          outblock, acc.at[pl.ds(sid * RPT + NBLK * z, NBLK)], usem))
    for dsc in zdescs:
      dsc.wait()
    plsc.subcore_barrier()

    da, db, dc = _fire_edges(0, 0)
    da.wait(); db.wait(); dc.wait()
    da, db, dc = _fire_edges(1, 1)
    da.wait(); db.wait(); dc.wait()
    _fire_gathers(l, 0, 0)
    _fire_gathers(l, 1, 1)

    def _quad(i, _):
      for u in range(4):
        s = 4 * i + u
        p = u % 2
        e = u
        e2 = (u + 2) % 4

        @pl.when(s + 2 < NSB)
        def _():
          _fire_edges(s + 2, e2)

        _wait_gathers(l, e, p)

        def _grp(eg, __):
          w16 = wb_[e][pl.ds(16 * eg, 16)]
          rowv = _iota16() + 16 * eg
          for dc_ in range(H):
            colv = jnp.full((16,), dc_, _i32)
            v = plsc.load_gather(rows_[p], [rowv, colv])
            plsc.store_scatter(rows_[p], [rowv, colv], v * w16)
          return __
        lax.fori_loop(0, SB // 16, _grp, None)

        for j in range(2):
          pltpu.sync_copy(rows_[p].at[pl.ds(128 * j, 128)],
                          acc.at[dstb_[e].at[j]], add=True)

        @pl.when(s + 2 < NSB)
        def _():
          _wait_edges(s + 2, e2)
          _fire_gathers(l, e2, p)
      return _
    lax.fori_loop(0, NSB // 4, _quad, None)
    plsc.subcore_barrier()

    cdescs = []
    for z in range(12):
      rr = sid * RPT + 256 * z
      cdescs.append(pltpu.async_copy(
          acc.at[pl.ds(rr, 256)], xs_h.at[l + 1, cid, pl.ds(rr, 256)], usem))
    cdescs.append(pltpu.async_copy(
        acc.at[pl.ds(sid * RPT + 3072, 128)],
        xs_h.at[l + 1, cid, pl.ds(sid * RPT + 3072, 128)], usem))
    for dsc in cdescs:
      dsc.wait()
    plsc.subcore_barrier()

  gidx = srcb0.at[pl.ds(0, 128)]
  for ridx, idx_h in enumerate((users_h, pos_h, neg_h)):
    for t in range(SPT // 128):
      s0 = sid * SPT + 128 * t
      pltpu.sync_copy(idx_h.at[pl.ds(s0, 128)], gidx)
      if ridx > 0:
        def _off(i, _):
          v = srcb0[pl.ds(16 * i, 16)]
          srcb0[pl.ds(16 * i, 16)] = v + NU
          return _
        lax.fori_loop(0, 8, _off, None)
      descs = []
      for l4 in range(NL + 1):
        descs.append(pltpu.async_copy(
            xs_h.at[l4, cid].at[gidx],
            rows_[l4 // 2].at[pl.ds(128 * (l4 % 2), 128)], gsem_[l4 // 2]))
      for dsc in descs:
        dsc.wait()

      for c in range(2):
        def _mrow(r, _):
          for h2 in range(2):
            sl = pl.ds(16 * h2, 16)
            rr = 64 * c + r
            v = (rows0[rr, sl] + rows0[128 + rr, sl]
                 + rows1[rr, sl] + rows1[128 + rr, sl]) * 0.25
            outblock[r, sl] = v
          return _
        lax.fori_loop(0, 64, _mrow, None)
        pltpu.sync_copy(outblock,
                        mean_h.at[ridx, cid, pl.ds(s0 + 64 * c, 64)])

  for ridx, (idx_h, tbl_h) in enumerate(((users_h, ue_h),
                                         (pos_h, ie_h),
                                         (neg_h, ie_h))):
    s0 = cid * (B // 2) + sid * 128
    pltpu.sync_copy(idx_h.at[pl.ds(s0, 128)], gidx)
    for c in range(2):
      pltpu.async_copy(tbl_h.at[srcb0.at[pl.ds(64 * c, 64)]],
                       ublock, usem).wait()
      pltpu.sync_copy(ublock, reg_h.at[ridx, pl.ds(s0 + 64 * c, 64)])


_sc_forward = pl.kernel(
    _sc_body,
    out_type=(
        jax.ShapeDtypeStruct((NL + 1, 2, NNP, H), _f32),
        jax.ShapeDtypeStruct((3, 2, B, H), _f32),
        jax.ShapeDtypeStruct((3, B, D), _f32),
    ),
    mesh=plsc.VectorSubcoreMesh(core_axis_name="c", subcore_axis_name="s",
                                num_cores=2, num_subcores=16),
    compiler_params=pltpu.CompilerParams(needs_layout_passes=False,
                                         use_tc_tiling_on_sc=False),
    scratch_types=[
        pltpu.VMEM_SHARED((NNP, H), _f32),
        pltpu.VMEM((3, D), _f32),
        pltpu.VMEM((8, D), _f32),
        pltpu.VMEM((11, D), _f32),
        pltpu.VMEM((NBLK, D), _f32),
        pltpu.VMEM((NBLK, H), _f32),
        pltpu.VMEM((SB, H), _f32),
        pltpu.VMEM((SB, H), _f32),
        pltpu.VMEM((SB,), _i32),
        pltpu.VMEM((SB,), _i32),
        pltpu.VMEM((SB,), _i32),
        pltpu.VMEM((SB,), _i32),
        pltpu.VMEM((2, 128), _i32),
        pltpu.VMEM((2, 128), _i32),
        pltpu.VMEM((2, 128), _i32),
        pltpu.VMEM((2, 128), _i32),
        pltpu.VMEM((SB,), _f32),
        pltpu.VMEM((SB,), _f32),
        pltpu.VMEM((SB,), _f32),
        pltpu.VMEM((SB,), _f32),
        pltpu.SemaphoreType.DMA,
        pltpu.SemaphoreType.DMA,
        pltpu.SemaphoreType.DMA,
        pltpu.SemaphoreType.DMA,
        pltpu.SemaphoreType.DMA,
        pltpu.SemaphoreType.DMA,
        pltpu.SemaphoreType.DMA,
    ],
)


def _loss_body(mean_ref, reg_ref, out_ref):
  u = mean_ref[0]
  pi = mean_ref[1]
  ni = mean_ref[2]
  ps = jnp.sum(u * pi, axis=(0, 2))
  ns = jnp.sum(u * ni, axis=(0, 2))
  x = ps - ns
  bpr = -jnp.mean(jnp.minimum(x, 0.0) - jnp.log1p(jnp.exp(-jnp.abs(x))))
  r = reg_ref[...]
  reg = jnp.sum(r * r) / B
  out_ref[...] = jnp.reshape(bpr + DECAY * reg, (1, 1))


_tc_loss = pl.pallas_call(
    _loss_body,
    out_shape=jax.ShapeDtypeStruct((1, 1), _f32),
)


@jax.jit
def kernel(users, pos_items, neg_items, edge_index, edge_weight,
           user_gender, user_age_bucket, item_cat,
           user_emb, item_emb, gender_emb, age_emb, cat_emb):
  dst = edge_index[0].astype(_i32)
  src = edge_index[1].astype(_i32)
  pad = NEP - NE
  pad_idx = jnp.arange(pad, dtype=_i32)
  src1 = jnp.concatenate([src, pad_idx % NN])
  dst1 = jnp.concatenate([dst, NN + pad_idx % (NNP - NN)])
  w1 = jnp.concatenate([edge_weight.astype(_f32), jnp.zeros((pad,), _f32)])
  dst2 = dst1.reshape(NEP // 128, 128)

  _, mean_rows, reg_rows = _sc_forward(
      users.astype(_i32), pos_items.astype(_i32), neg_items.astype(_i32),
      src1, dst2, w1,
      user_gender.astype(_i32), user_age_bucket.astype(_i32),
      item_cat.astype(_i32),
      user_emb.astype(_f32), item_emb.astype(_f32),
      gender_emb.astype(_f32), age_emb.astype(_f32), cat_emb.astype(_f32))

  loss = _tc_loss(mean_rows, reg_rows)
  return jnp.reshape(loss, ())


# speedup vs baseline: 14.7168x; 8.2188x over previous
import functools

import jax
import jax.numpy as jnp
from jax import lax
from jax.experimental import pallas as pl
from jax.experimental.pallas import tpu as pltpu
from jax.experimental.pallas import tpu_sc as plsc

NU = 25000
NI = 25000
NN = NU + NI
NNP = 51200
NE = 800000
NEP = 819200
D = 64
H = 32
B = 4096
NL = 3
DECAY = 1e-4

NBLK = 64
NUB = 391
EPT = NEP // 16
SB = 256
NSB = EPT // SB
RPT = NNP // 16
SPT = B // 16

_f32 = jnp.float32
_i32 = jnp.int32


def _iota16():
  return lax.iota(_i32, 16)


def _sc_body(users_h, pos_h, neg_h, src_h, dst2_h, w_h,
             ug_h, ua_h, ic_h, ue_h, ie_h, ge_h, ae_h, ce_h,
             xs_h, mean_h, reg_h,
             acc, gtab, atab, ctab, ublock, outblock, rows0, rows1,
             srcb0, srcb1, srcb2, srcb3, dstb0, dstb1, dstb2, dstb3,
             wb0, wb1, wb2, wb3,
             esem0, esem1, esem2, esem3, gsem0, gsem1, usem):
  cid = lax.axis_index("c")
  sid = lax.axis_index("s")
  col_base = cid * H
  cb16 = col_base
  rows_ = (rows0, rows1)
  srcb_ = (srcb0, srcb1, srcb2, srcb3)
  dstb_ = (dstb0, dstb1, dstb2, dstb3)
  wb_ = (wb0, wb1, wb2, wb3)
  esem_ = (esem0, esem1, esem2, esem3)
  gsem_ = (gsem0, gsem1)

  pltpu.sync_copy(ge_h, gtab)
  pltpu.sync_copy(ae_h, atab)
  pltpu.sync_copy(ce_h, ctab)

  def _build_user(b):
    n0 = jnp.minimum(NBLK * b, NU - NBLK)
    d1 = pltpu.async_copy(ue_h.at[pl.ds(n0, NBLK)], ublock, usem)
    d2 = pltpu.async_copy(ug_h.at[pl.ds(n0, NBLK)], gidx_b, usem)
    d3 = pltpu.async_copy(ua_h.at[pl.ds(n0, NBLK)], aidx_b, usem)
    d1.wait(); d2.wait(); d3.wait()

    def _grp(g, _):
      gv = gidx_b[pl.ds(16 * g, 16)]
      av = aidx_b[pl.ds(16 * g, 16)]
      for j in range(16):
        n = 16 * g + j
        gj = gv[j]
        aj = av[j]
        for h2 in range(2):
          sl = pl.ds(16 * h2, 16)
          outblock[n, sl] = (ublock[n, pl.ds(cb16 + 16 * h2, 16)]
                             + gtab[gj, pl.ds(cb16 + 16 * h2, 16)]
                             + atab[aj, pl.ds(cb16 + 16 * h2, 16)])
      return _
    lax.fori_loop(0, NBLK // 16, _grp, None)
    pltpu.sync_copy(outblock, xs_h.at[0, cid, pl.ds(n0, NBLK)])

  def _build_item(b):
    n0 = jnp.minimum(NBLK * b, NI - NBLK)
    d1 = pltpu.async_copy(ie_h.at[pl.ds(n0, NBLK)], ublock, usem)
    d2 = pltpu.async_copy(ic_h.at[pl.ds(n0, NBLK)], gidx_b, usem)
    d1.wait(); d2.wait()

    def _grp(g, _):
      cv = gidx_b[pl.ds(16 * g, 16)]
      for j in range(16):
        n = 16 * g + j
        cj = cv[j]
        for h2 in range(2):
          sl = pl.ds(16 * h2, 16)
          outblock[n, sl] = (ublock[n, pl.ds(cb16 + 16 * h2, 16)]
                             + ctab[cj, pl.ds(cb16 + 16 * h2, 16)])
      return _
    lax.fori_loop(0, NBLK // 16, _grp, None)
    pltpu.sync_copy(outblock, xs_h.at[0, cid, pl.ds(NU + n0, NBLK)])

  gidx_b = srcb0.at[pl.ds(0, NBLK)]
  aidx_b = srcb0.at[pl.ds(NBLK, NBLK)]

  def _build_k(k, _):
    b = sid + 16 * k

    @pl.when(b < NUB)
    def _():
      _build_user(b)
      _build_item(b)
    return _
  lax.fori_loop(0, 25, _build_k, None)

  plsc.subcore_barrier()

  def _zb(r, _):
    outblock[r, pl.ds(0, 16)] = jnp.zeros((16,), _f32)
    outblock[r, pl.ds(16, 16)] = jnp.zeros((16,), _f32)
    return _
  lax.fori_loop(0, NBLK, _zb, None)

  def _fire_edges(s, e):
    e0 = sid * EPT + SB * s
    r0 = sid * (EPT // 128) + 2 * s
    da = pltpu.async_copy(src_h.at[pl.ds(e0, SB)], srcb_[e], esem_[e])
    db = pltpu.async_copy(dst2_h.at[pl.ds(r0, 2)], dstb_[e], esem_[e])
    dc = pltpu.async_copy(w_h.at[pl.ds(e0, SB)], wb_[e], esem_[e])
    return da, db, dc

  def _wait_edges(s, e):
    e0 = sid * EPT + SB * s
    r0 = sid * (EPT // 128) + 2 * s
    pltpu.make_async_copy(src_h.at[pl.ds(e0, SB)], srcb_[e], esem_[e]).wait()
    pltpu.make_async_copy(dst2_h.at[pl.ds(r0, 2)], dstb_[e], esem_[e]).wait()
    pltpu.make_async_copy(w_h.at[pl.ds(e0, SB)], wb_[e], esem_[e]).wait()

  def _fire_gathers(l, e, p):
    for j in range(2):
      pltpu.async_copy(
          xs_h.at[l, cid].at[srcb_[e].at[pl.ds(128 * j, 128)]],
          rows_[p].at[pl.ds(128 * j, 128)], gsem_[p])

  def _wait_gathers(l, e, p):
    for j in range(2):
      pltpu.make_async_copy(
          xs_h.at[l, cid].at[srcb_[e].at[pl.ds(128 * j, 128)]],
          rows_[p].at[pl.ds(128 * j, 128)], gsem_[p]).wait()

  for l in range(NL):
    zdescs = []
    for z in range(RPT // NBLK):
      zdescs.append(pltpu.async_copy(
          outblock, acc.at[pl.ds(sid * RPT + NBLK * z, NBLK)], usem))
    for dsc in zdescs:
      dsc.wait()
    plsc.subcore_barrier()

    da, db, dc = _fire_edges(0, 0)
    da.wait(); db.wait(); dc.wait()
    da, db, dc = _fire_edges(1, 1)
    da.wait(); db.wait(); dc.wait()
    _fire_gathers(l, 0, 0)
    _fire_gathers(l, 1, 1)

    def _quad(i, _):
      for u in range(4):
        s = 4 * i + u
        p = u % 2
        e = u
        e2 = (u + 2) % 4

        @pl.when(s + 2 < NSB)
        def _():
          _fire_edges(s + 2, e2)

        _wait_gathers(l, e, p)

        def _grp(eg, __):
          w16 = wb_[e][pl.ds(16 * eg, 16)]
          base = 16 * eg
          for j in range(16):
            wj = jnp.broadcast_to(w16[j], (16,))
            for h2 in range(2):
              sl = pl.ds(16 * h2, 16)
              rows_[p][base + j, sl] = rows_[p][base + j, sl] * wj
          return __
        lax.fori_loop(0, SB // 16, _grp, None)

        for j in range(2):
          pltpu.sync_copy(rows_[p].at[pl.ds(128 * j, 128)],
                          acc.at[dstb_[e].at[j]], add=True)

        @pl.when(s + 2 < NSB)
        def _():
          _wait_edges(s + 2, e2)
          _fire_gathers(l, e2, p)
      return _
    lax.fori_loop(0, NSB // 4, _quad, None)
    plsc.subcore_barrier()

    cdescs = []
    for z in range(12):
      rr = sid * RPT + 256 * z
      cdescs.append(pltpu.async_copy(
          acc.at[pl.ds(rr, 256)], xs_h.at[l + 1, cid, pl.ds(rr, 256)], usem))
    cdescs.append(pltpu.async_copy(
        acc.at[pl.ds(sid * RPT + 3072, 128)],
        xs_h.at[l + 1, cid, pl.ds(sid * RPT + 3072, 128)], usem))
    for dsc in cdescs:
      dsc.wait()
    plsc.subcore_barrier()

  gidx = srcb0.at[pl.ds(0, 128)]
  for ridx, idx_h in enumerate((users_h, pos_h, neg_h)):
    for t in range(SPT // 128):
      s0 = sid * SPT + 128 * t
      pltpu.sync_copy(idx_h.at[pl.ds(s0, 128)], gidx)
      if ridx > 0:
        def _off(i, _):
          v = srcb0[pl.ds(16 * i, 16)]
          srcb0[pl.ds(16 * i, 16)] = v + NU
          return _
        lax.fori_loop(0, 8, _off, None)
      descs = []
      for l4 in range(NL + 1):
        descs.append(pltpu.async_copy(
            xs_h.at[l4, cid].at[gidx],
            rows_[l4 // 2].at[pl.ds(128 * (l4 % 2), 128)], gsem_[l4 // 2]))
      for dsc in descs:
        dsc.wait()

      for c in range(2):
        def _mrow(r, _):
          for h2 in range(2):
            sl = pl.ds(16 * h2, 16)
            rr = 64 * c + r
            v = (rows0[rr, sl] + rows0[128 + rr, sl]
                 + rows1[rr, sl] + rows1[128 + rr, sl]) * 0.25
            outblock[r, sl] = v
          return _
        lax.fori_loop(0, 64, _mrow, None)
        pltpu.sync_copy(outblock,
                        mean_h.at[ridx, cid, pl.ds(s0 + 64 * c, 64)])

  for ridx, (idx_h, tbl_h) in enumerate(((users_h, ue_h),
                                         (pos_h, ie_h),
                                         (neg_h, ie_h))):
    s0 = cid * (B // 2) + sid * 128
    pltpu.sync_copy(idx_h.at[pl.ds(s0, 128)], gidx)
    for c in range(2):
      pltpu.async_copy(tbl_h.at[srcb0.at[pl.ds(64 * c, 64)]],
                       ublock, usem).wait()
      pltpu.sync_copy(ublock, reg_h.at[ridx, pl.ds(s0 + 64 * c, 64)])


_sc_forward = pl.kernel(
    _sc_body,
    out_type=(
        jax.ShapeDtypeStruct((NL + 1, 2, NNP, H), _f32),
        jax.ShapeDtypeStruct((3, 2, B, H), _f32),
        jax.ShapeDtypeStruct((3, B, D), _f32),
    ),
    mesh=plsc.VectorSubcoreMesh(core_axis_name="c", subcore_axis_name="s",
                                num_cores=2, num_subcores=16),
    compiler_params=pltpu.CompilerParams(needs_layout_passes=False,
                                         use_tc_tiling_on_sc=False),
    scratch_types=[
        pltpu.VMEM_SHARED((NNP, H), _f32),
        pltpu.VMEM((3, D), _f32),
        pltpu.VMEM((8, D), _f32),
        pltpu.VMEM((11, D), _f32),
        pltpu.VMEM((NBLK, D), _f32),
        pltpu.VMEM((NBLK, H), _f32),
        pltpu.VMEM((SB, H), _f32),
        pltpu.VMEM((SB, H), _f32),
        pltpu.VMEM((SB,), _i32),
        pltpu.VMEM((SB,), _i32),
        pltpu.VMEM((SB,), _i32),
        pltpu.VMEM((SB,), _i32),
        pltpu.VMEM((2, 128), _i32),
        pltpu.VMEM((2, 128), _i32),
        pltpu.VMEM((2, 128), _i32),
        pltpu.VMEM((2, 128), _i32),
        pltpu.VMEM((SB,), _f32),
        pltpu.VMEM((SB,), _f32),
        pltpu.VMEM((SB,), _f32),
        pltpu.VMEM((SB,), _f32),
        pltpu.SemaphoreType.DMA,
        pltpu.SemaphoreType.DMA,
        pltpu.SemaphoreType.DMA,
        pltpu.SemaphoreType.DMA,
        pltpu.SemaphoreType.DMA,
        pltpu.SemaphoreType.DMA,
        pltpu.SemaphoreType.DMA,
    ],
)


def _loss_body(mean_ref, reg_ref, out_ref):
  u = mean_ref[0]
  pi = mean_ref[1]
  ni = mean_ref[2]
  ps = jnp.sum(u * pi, axis=(0, 2))
  ns = jnp.sum(u * ni, axis=(0, 2))
  x = ps - ns
  bpr = -jnp.mean(jnp.minimum(x, 0.0) - jnp.log1p(jnp.exp(-jnp.abs(x))))
  r = reg_ref[...]
  reg = jnp.sum(r * r) / B
  out_ref[...] = jnp.reshape(bpr + DECAY * reg, (1, 1))


_tc_loss = pl.pallas_call(
    _loss_body,
    out_shape=jax.ShapeDtypeStruct((1, 1), _f32),
)


@jax.jit
def kernel(users, pos_items, neg_items, edge_index, edge_weight,
           user_gender, user_age_bucket, item_cat,
           user_emb, item_emb, gender_emb, age_emb, cat_emb):
  dst = edge_index[0].astype(_i32)
  src = edge_index[1].astype(_i32)
  pad = NEP - NE
  pad_idx = jnp.arange(pad, dtype=_i32)
  src1 = jnp.concatenate([src, pad_idx % NN])
  dst1 = jnp.concatenate([dst, NN + pad_idx % (NNP - NN)])
  w1 = jnp.concatenate([edge_weight.astype(_f32), jnp.zeros((pad,), _f32)])
  dst2 = dst1.reshape(NEP // 128, 128)

  _, mean_rows, reg_rows = _sc_forward(
      users.astype(_i32), pos_items.astype(_i32), neg_items.astype(_i32),
      src1, dst2, w1,
      user_gender.astype(_i32), user_age_bucket.astype(_i32),
      item_cat.astype(_i32),
      user_emb.astype(_f32), item_emb.astype(_f32),
      gender_emb.astype(_f32), age_emb.astype(_f32), cat_emb.astype(_f32))

  loss = _tc_loss(mean_rows, reg_rows)
  return jnp.reshape(loss, ())
